# baseline (speedup 1.0000x reference)
import functools

import jax
import jax.numpy as jnp
from jax import lax
from jax.experimental import pallas as pl
from jax.experimental.pallas import tpu as pltpu
from jax.experimental.pallas import tpu_sc as plsc

N = 10000
E = 320000
D = 128

NC = 2
NS = 16
NW = NC * NS
EPW = E // NW
NV = EPW // 16
CH = 80
NCHUNK = EPW // CH
RPT = 624

_NEG = -3.0e38

_mesh = plsc.VectorSubcoreMesh(core_axis_name="c", subcore_axis_name="s")


def _wid():
    return lax.axis_index("s") * NC + lax.axis_index("c")


def _seg_reduce_scatter(ref, r, val, op):
    k2, v2 = plsc.sort_key_val(r, val)
    ii = lax.iota(jnp.int32, 16)
    for d in (1, 2, 4, 8):
        idx = jnp.maximum(ii - d, 0)
        pk = k2.at[idx].get(mode="promise_in_bounds")
        pv = v2.at[idx].get(mode="promise_in_bounds")
        take = (ii >= d) & (pk == k2)
        v2 = jnp.where(take, op(v2, pv), v2)
    nk = k2.at[jnp.minimum(ii + 1, 15)].get(mode="promise_in_bounds")
    last = (ii == 15) | (nk != k2)
    cur = plsc.load_gather(ref, [k2])
    plsc.store_scatter(ref, [k2], op(cur, v2), mask=last)


def _sc_a_body(row_h, col_h, a1_h, a2_h, v_h, mpart_h,
               row_v, col_v, a1_v, a2_v, m_v, v_v):
    wid = _wid()
    base = wid * EPW
    pltpu.sync_copy(a1_h, a1_v)
    pltpu.sync_copy(a2_h, a2_v)
    pltpu.sync_copy(row_h.at[pl.ds(base, EPW)], row_v)
    pltpu.sync_copy(col_h.at[pl.ds(base, EPW)], col_v)

    def init(i, _):
        m_v[pl.ds(i * 16, 16)] = jnp.full((16,), _NEG, jnp.float32)
        return _
    lax.fori_loop(0, NV, init, None)

    def body(i, _):
        sl = pl.ds(i * 16, 16)
        r = row_v[sl]
        c = col_v[sl]
        g1 = plsc.load_gather(a1_v, [r])
        g2 = plsc.load_gather(a2_v, [c])
        v = g1 + g2
        v = jnp.where(v > 0, v, v * jnp.float32(0.01))
        v_v[sl] = v
        _seg_reduce_scatter(m_v, r, v, jnp.maximum)
        return _
    lax.fori_loop(0, NV, body, None)

    pltpu.sync_copy(v_v, v_h.at[pl.ds(base, EPW)])
    pltpu.sync_copy(m_v, mpart_h.at[pl.ds(wid * N, N)])


_sc_a = pl.kernel(
    _sc_a_body,
    out_type=(jax.ShapeDtypeStruct((E,), jnp.float32),
              jax.ShapeDtypeStruct((NW * N,), jnp.float32)),
    mesh=_mesh,
    compiler_params=pltpu.CompilerParams(needs_layout_passes=False),
    scratch_types=[
        pltpu.VMEM((EPW,), jnp.int32),
        pltpu.VMEM((EPW,), jnp.int32),
        pltpu.VMEM((N,), jnp.float32),
        pltpu.VMEM((N,), jnp.float32),
        pltpu.VMEM((N,), jnp.float32),
        pltpu.VMEM((EPW,), jnp.float32),
    ],
)


def _sc_c_body(row_h, v_h, mpart_h, ev_h, spart_h,
               row_v, v_v, m_v, s_v, mb, mt, macc, msh, rsem):
    wid = _wid()
    sid = lax.axis_index("s")
    base = wid * EPW

    rbase = sid * RPT
    for w in range(NW):
        pltpu.async_copy(mpart_h.at[pl.ds(w * N + rbase, RPT)],
                         mb.at[pl.ds(w * RPT, RPT)], rsem)
    pltpu.make_async_copy(mpart_h.at[pl.ds(0, N * NW)], mb, rsem).wait()

    def mred(j, _):
        acc = mb[pl.ds(j * 16, 16)]
        for w in range(1, NW):
            acc = jnp.maximum(acc, mb[pl.ds(w * RPT + j * 16, 16)])
        macc[pl.ds(j * 16, 16)] = acc
        return _
    lax.fori_loop(0, RPT // 16, mred, None)
    pltpu.sync_copy(macc, msh.at[pl.ds(rbase, RPT)])

    @pl.when(sid == NS - 1)
    def _():
        tb = NS * RPT
        for w in range(NW):
            pltpu.async_copy(mpart_h.at[pl.ds(w * N + tb, N - tb)],
                             mt.at[pl.ds(w * 16, 16)], rsem)
        pltpu.make_async_copy(mpart_h.at[pl.ds(0, (N - tb) * NW)],
                              mt, rsem).wait()
        acc = mt[pl.ds(0, 16)]
        for w in range(1, NW):
            acc = jnp.maximum(acc, mt[pl.ds(w * 16, 16)])
        macc[pl.ds(0, 16)] = acc
        pltpu.sync_copy(macc.at[pl.ds(0, 16)], msh.at[pl.ds(tb, N - tb)])
    plsc.subcore_barrier()
    pltpu.sync_copy(msh, m_v)

    pltpu.sync_copy(row_h.at[pl.ds(base, EPW)], row_v)
    pltpu.sync_copy(v_h.at[pl.ds(base, EPW)], v_v)

    def init(i, _):
        s_v[pl.ds(i * 16, 16)] = jnp.zeros((16,), jnp.float32)
        return _
    lax.fori_loop(0, NV, init, None)

    def body(i, _):
        sl = pl.ds(i * 16, 16)
        r = row_v[sl]
        v = v_v[sl]
        mm = plsc.load_gather(m_v, [r])
        ev = jnp.exp(v - mm)
        v_v[sl] = ev
        plsc.addupdate_scatter(s_v, [r], ev)
        return _
    lax.fori_loop(0, NV, body, None)

    pltpu.sync_copy(v_v, ev_h.at[pl.ds(base, EPW)])
    pltpu.sync_copy(s_v, spart_h.at[pl.ds(wid * N, N)])


_sc_c = pl.kernel(
    _sc_c_body,
    out_type=(jax.ShapeDtypeStruct((E,), jnp.float32),
              jax.ShapeDtypeStruct((NW * N,), jnp.float32)),
    mesh=_mesh,
    compiler_params=pltpu.CompilerParams(needs_layout_passes=False),
    scratch_types=[
        pltpu.VMEM((EPW,), jnp.int32),
        pltpu.VMEM((EPW,), jnp.float32),
        pltpu.VMEM((N,), jnp.float32),
        pltpu.VMEM((N,), jnp.float32),
        pltpu.VMEM((NW * RPT,), jnp.float32),
        pltpu.VMEM((NW * 16,), jnp.float32),
        pltpu.VMEM((RPT,), jnp.float32),
        pltpu.VMEM_SHARED((N,), jnp.float32),
        pltpu.SemaphoreType.DMA,
    ],
)


def _sc_e_body(row_h, col_h, ev_h, featp_h, outp_h,
               col_v, gb0, gb1, sb0, sb1, rb0, rb1, eb0, eb1, out_sh,
               gsem0, gsem1, ssem0, ssem1):
    cid = lax.axis_index("c")
    sid = lax.axis_index("s")
    wid = sid * NC + cid
    base = wid * EPW
    gbufs = (gb0, gb1)
    sbufs = (sb0, sb1)
    rbufs = (rb0, rb1)
    ebufs = (eb0, eb1)
    gsems = (gsem0, gsem1)
    ssems = (ssem0, ssem1)

    pltpu.sync_copy(col_h.at[pl.ds(base, EPW)], col_v)

    def zrow(i, _):
        for cb in range(8):
            sb0[i, pl.ds(cb * 16, 16)] = jnp.zeros((16,), jnp.float32)
        return _
    lax.fori_loop(0, 16, zrow, None)
    rbase = sid * RPT

    def zcopy(z, _):
        pltpu.sync_copy(sb0.at[pl.ds(0, 16)],
                        out_sh.at[pl.ds(rbase + z * 16, 16)])
        return _
    lax.fori_loop(0, RPT // 16, zcopy, None)

    @pl.when(sid == NS - 1)
    def _():
        pltpu.sync_copy(sb0.at[pl.ds(0, 16)],
                        out_sh.at[pl.ds(NS * RPT, N - NS * RPT)])
    plsc.subcore_barrier()

    def issue_gather(c, sl):
        pltpu.async_copy(
            featp_h.at[col_v.at[pl.ds(c * CH, CH)]], gbufs[sl], gsems[sl])
        pltpu.async_copy(row_h.at[pl.ds(base + c * CH, CH)], rbufs[sl],
                         gsems[sl])
        pltpu.async_copy(ev_h.at[pl.ds(base + c * CH, CH)], ebufs[sl],
                         gsems[sl])

    def drain_gather(sl):
        pltpu.make_async_copy(
            featp_h.at[pl.ds(0, CH)], gbufs[sl], gsems[sl]).wait()
        pltpu.make_async_copy(
            row_h.at[pl.ds(0, CH)], rbufs[sl], gsems[sl]).wait()
        pltpu.make_async_copy(
            ev_h.at[pl.ds(0, CH)], ebufs[sl], gsems[sl]).wait()

    def drain_scatter(sl):
        pltpu.make_async_copy(
            ev_h.at[pl.ds(0, CH * D)], sbufs[sl], ssems[sl]).wait()

    _HI = jnp.int32(-65536)

    def process(c, sl):
        drain_gather(sl)

        @pl.when(c >= 2)
        def _():
            drain_scatter(sl)

        gb = gbufs[sl]
        sb = sbufs[sl]
        rb = rbufs[sl]
        eb = ebufs[sl]

        def kstep(k, _):
            av16 = eb[pl.ds(k * 16, 16)]
            for ri in range(16):
                att = av16[ri]
                rr = k * 16 + ri
                for cg in range(4):
                    xi = gb[rr, pl.ds(cg * 16, 16)]
                    lo = plsc.bitcast(lax.shift_left(xi, 16), jnp.float32)
                    hi = plsc.bitcast(xi & _HI, jnp.float32)
                    sb[rr, pl.ds(cg * 16, 16)] = lo * att
                    sb[rr, pl.ds(64 + cg * 16, 16)] = hi * att
            ridx = rb[pl.ds(k * 16, 16)]
            pltpu.async_copy(sb.at[pl.ds(k * 16, 16)], out_sh.at[ridx],
                             ssems[sl], add=True)
            return _
        lax.fori_loop(0, CH // 16, kstep, None)

        @pl.when(c + 2 < NCHUNK)
        def _():
            issue_gather(c + 2, sl)

    issue_gather(0, 0)
    issue_gather(1, 1)

    def outer(gg, _):
        process(gg * 2, 0)
        process(gg * 2 + 1, 1)
        return _
    lax.fori_loop(0, NCHUNK // 2, outer, None)
    process(NCHUNK - 1, 0)

    drain_scatter(1)
    drain_scatter(0)
    plsc.subcore_barrier()
    pltpu.sync_copy(out_sh.at[pl.ds(rbase, RPT)],
                    outp_h.at[cid, pl.ds(rbase, RPT)])

    @pl.when(sid == NS - 1)
    def _():
        pltpu.sync_copy(out_sh.at[pl.ds(NS * RPT, N - NS * RPT)],
                        outp_h.at[cid, pl.ds(NS * RPT, N - NS * RPT)])


_sc_e = pl.kernel(
    _sc_e_body,
    out_type=jax.ShapeDtypeStruct((NC, N, D), jnp.float32),
    mesh=_mesh,
    compiler_params=pltpu.CompilerParams(needs_layout_passes=False, use_tc_tiling_on_sc=False),
    scratch_types=[
        pltpu.VMEM((EPW,), jnp.int32),
        pltpu.VMEM((CH, D // 2), jnp.int32),
        pltpu.VMEM((CH, D // 2), jnp.int32),
        pltpu.VMEM((CH, D), jnp.float32),
        pltpu.VMEM((CH, D), jnp.float32),
        pltpu.VMEM((CH,), jnp.int32),
        pltpu.VMEM((CH,), jnp.int32),
        pltpu.VMEM((CH,), jnp.float32),
        pltpu.VMEM((CH,), jnp.float32),
        pltpu.VMEM_SHARED((N, D), jnp.float32),
        pltpu.SemaphoreType.DMA,
        pltpu.SemaphoreType.DMA,
        pltpu.SemaphoreType.DMA,
        pltpu.SemaphoreType.DMA,
    ],
)


_BN = 1000


def _proj_body(x_ref, w_ref, b_ref, aw_ref, ab_ref, feat_ref, a12_ref):
    x = x_ref[...]
    feat = lax.dot_general(x, w_ref[...], (((1,), (1,)), ((), ())),
                           preferred_element_type=jnp.float32)
    feat = feat + b_ref[...]
    feat_ref[...] = feat
    a12_ref[...] = lax.dot_general(feat, aw_ref[...], (((1,), (0,)), ((), ())),
                                   preferred_element_type=jnp.float32) + ab_ref[...]


_proj = pl.pallas_call(
    _proj_body,
    grid=(N // _BN,),
    in_specs=[
        pl.BlockSpec((_BN, D), lambda i: (i, 0)),
        pl.BlockSpec((D, D), lambda i: (0, 0)),
        pl.BlockSpec((1, D), lambda i: (0, 0)),
        pl.BlockSpec((D, 8), lambda i: (0, 0)),
        pl.BlockSpec((1, 8), lambda i: (0, 0)),
    ],
    out_specs=[
        pl.BlockSpec((_BN, D), lambda i: (i, 0)),
        pl.BlockSpec((_BN, 8), lambda i: (i, 0)),
    ],
    out_shape=[
        jax.ShapeDtypeStruct((N, D), jnp.float32),
        jax.ShapeDtypeStruct((N, 8), jnp.float32),
    ],
)


def _make_reduce(op):
    def body(p_ref, o_ref):
        o_ref[...] = jnp.broadcast_to(op(p_ref[...], axis=0, keepdims=True),
                                      (8, N))
    return pl.pallas_call(
        body,
        out_shape=jax.ShapeDtypeStruct((8, N), jnp.float32),
    )


_maxred = _make_reduce(jnp.max)
_sumred = _make_reduce(jnp.sum)


def _add_body(p_ref, sp_ref, o_ref):
    s = jnp.sum(sp_ref[...], axis=0)
    s = jnp.where(s == 0, jnp.float32(1), s)
    o_ref[...] = (p_ref[0] + p_ref[1]) / s[:, None]


_final_add = pl.pallas_call(
    _add_body,
    out_shape=jax.ShapeDtypeStruct((N, D), jnp.float32),
)


def kernel(features, adj_indices, W, b, a1_w, a1_b, a2_w, a2_b):
    row = adj_indices[0]
    col = adj_indices[1]
    aw = jnp.pad(jnp.stack([a1_w, a2_w], axis=1), ((0, 0), (0, 6)))
    ab = jnp.pad(jnp.stack([jnp.asarray(a1_b, jnp.float32),
                            jnp.asarray(a2_b, jnp.float32)])[None, :],
                 ((0, 0), (0, 6)))
    feat, a12 = _proj(features, W, b[None, :], aw, ab)
    a1 = a12[:, 0]
    a2 = a12[:, 1]
    v, mpart = _sc_a(row, col, a1, a2)
    ev, spart = _sc_c(row, v, mpart)
    fb = feat.astype(jnp.bfloat16)
    lo = jax.lax.bitcast_convert_type(fb[:, :64], jnp.uint16).astype(jnp.uint32)
    hi = jax.lax.bitcast_convert_type(fb[:, 64:], jnp.uint16).astype(jnp.uint32)
    featp = jax.lax.bitcast_convert_type(lo | (hi << 16), jnp.int32)
    outp = _sc_e(row, col, ev, featp)
    return _final_add(outp, spart.reshape(NW, N))


# speedup vs baseline: 1.5907x; 1.5907x over previous
import functools

import jax
import jax.numpy as jnp
from jax import lax
from jax.experimental import pallas as pl
from jax.experimental.pallas import tpu as pltpu
from jax.experimental.pallas import tpu_sc as plsc

N = 10000
E = 320000
D = 128

NC = 2
NS = 16
NW = NC * NS
EPW = E // NW
NV = EPW // 16
CH = 80
NCHUNK = EPW // CH
RPT = 624

_NEG = -3.0e38

_mesh = plsc.VectorSubcoreMesh(core_axis_name="c", subcore_axis_name="s")


def _wid():
    return lax.axis_index("s") * NC + lax.axis_index("c")


def _seg_reduce_scatter(ref, r, val, op):
    k2, v2 = plsc.sort_key_val(r, val)
    ii = lax.iota(jnp.int32, 16)
    for d in (1, 2, 4, 8):
        idx = jnp.maximum(ii - d, 0)
        pk = k2.at[idx].get(mode="promise_in_bounds")
        pv = v2.at[idx].get(mode="promise_in_bounds")
        take = (ii >= d) & (pk == k2)
        v2 = jnp.where(take, op(v2, pv), v2)
    nk = k2.at[jnp.minimum(ii + 1, 15)].get(mode="promise_in_bounds")
    last = (ii == 15) | (nk != k2)
    cur = plsc.load_gather(ref, [k2])
    plsc.store_scatter(ref, [k2], op(cur, v2), mask=last)


def _sc_a_body(row_h, col_h, a1_h, a2_h, v_h, mpart_h,
               row_v, col_v, a1_v, a2_v, m_v, v_v):
    wid = _wid()
    base = wid * EPW
    pltpu.sync_copy(a1_h, a1_v)
    pltpu.sync_copy(a2_h, a2_v)
    pltpu.sync_copy(row_h.at[pl.ds(base, EPW)], row_v)
    pltpu.sync_copy(col_h.at[pl.ds(base, EPW)], col_v)

    def init(i, _):
        m_v[pl.ds(i * 16, 16)] = jnp.full((16,), _NEG, jnp.float32)
        return _
    lax.fori_loop(0, NV, init, None)

    def body(i, _):
        sl = pl.ds(i * 16, 16)
        r = row_v[sl]
        c = col_v[sl]
        g1 = plsc.load_gather(a1_v, [r])
        g2 = plsc.load_gather(a2_v, [c])
        v = g1 + g2
        v = jnp.where(v > 0, v, v * jnp.float32(0.01))
        v_v[sl] = v
        _seg_reduce_scatter(m_v, r, v, jnp.maximum)
        return _
    lax.fori_loop(0, NV, body, None)

    pltpu.sync_copy(v_v, v_h.at[pl.ds(base, EPW)])
    pltpu.sync_copy(m_v, mpart_h.at[pl.ds(wid * N, N)])


_sc_a = pl.kernel(
    _sc_a_body,
    out_type=(jax.ShapeDtypeStruct((E,), jnp.float32),
              jax.ShapeDtypeStruct((NW * N,), jnp.float32)),
    mesh=_mesh,
    compiler_params=pltpu.CompilerParams(needs_layout_passes=False),
    scratch_types=[
        pltpu.VMEM((EPW,), jnp.int32),
        pltpu.VMEM((EPW,), jnp.int32),
        pltpu.VMEM((N,), jnp.float32),
        pltpu.VMEM((N,), jnp.float32),
        pltpu.VMEM((N,), jnp.float32),
        pltpu.VMEM((EPW,), jnp.float32),
    ],
)


def _sc_c_body(row_h, v_h, mpart_h, ev_h, spart_h,
               row_v, v_v, m_v, s_v, mb, mt, macc, msh, rsem):
    wid = _wid()
    sid = lax.axis_index("s")
    base = wid * EPW

    rbase = sid * RPT
    for w in range(NW):
        pltpu.async_copy(mpart_h.at[pl.ds(w * N + rbase, RPT)],
                         mb.at[pl.ds(w * RPT, RPT)], rsem)
    pltpu.make_async_copy(mpart_h.at[pl.ds(0, N * NW)], mb, rsem).wait()

    def mred(j, _):
        acc = mb[pl.ds(j * 16, 16)]
        for w in range(1, NW):
            acc = jnp.maximum(acc, mb[pl.ds(w * RPT + j * 16, 16)])
        macc[pl.ds(j * 16, 16)] = acc
        return _
    lax.fori_loop(0, RPT // 16, mred, None)
    pltpu.sync_copy(macc, msh.at[pl.ds(rbase, RPT)])

    @pl.when(sid == NS - 1)
    def _():
        tb = NS * RPT
        for w in range(NW):
            pltpu.async_copy(mpart_h.at[pl.ds(w * N + tb, N - tb)],
                             mt.at[pl.ds(w * 16, 16)], rsem)
        pltpu.make_async_copy(mpart_h.at[pl.ds(0, (N - tb) * NW)],
                              mt, rsem).wait()
        acc = mt[pl.ds(0, 16)]
        for w in range(1, NW):
            acc = jnp.maximum(acc, mt[pl.ds(w * 16, 16)])
        macc[pl.ds(0, 16)] = acc
        pltpu.sync_copy(macc.at[pl.ds(0, 16)], msh.at[pl.ds(tb, N - tb)])
    plsc.subcore_barrier()
    pltpu.sync_copy(msh, m_v)

    pltpu.sync_copy(row_h.at[pl.ds(base, EPW)], row_v)
    pltpu.sync_copy(v_h.at[pl.ds(base, EPW)], v_v)

    def init(i, _):
        s_v[pl.ds(i * 16, 16)] = jnp.zeros((16,), jnp.float32)
        return _
    lax.fori_loop(0, NV, init, None)

    def body(i, _):
        sl = pl.ds(i * 16, 16)
        r = row_v[sl]
        v = v_v[sl]
        mm = plsc.load_gather(m_v, [r])
        ev = jnp.exp(v - mm)
        v_v[sl] = ev
        plsc.addupdate_scatter(s_v, [r], ev)
        return _
    lax.fori_loop(0, NV, body, None)

    pltpu.sync_copy(v_v, ev_h.at[pl.ds(base, EPW)])
    pltpu.sync_copy(s_v, spart_h.at[pl.ds(wid * N, N)])


_sc_c = pl.kernel(
    _sc_c_body,
    out_type=(jax.ShapeDtypeStruct((E,), jnp.float32),
              jax.ShapeDtypeStruct((NW * N,), jnp.float32)),
    mesh=_mesh,
    compiler_params=pltpu.CompilerParams(needs_layout_passes=False),
    scratch_types=[
        pltpu.VMEM((EPW,), jnp.int32),
        pltpu.VMEM((EPW,), jnp.float32),
        pltpu.VMEM((N,), jnp.float32),
        pltpu.VMEM((N,), jnp.float32),
        pltpu.VMEM((NW * RPT,), jnp.float32),
        pltpu.VMEM((NW * 16,), jnp.float32),
        pltpu.VMEM((RPT,), jnp.float32),
        pltpu.VMEM_SHARED((N,), jnp.float32),
        pltpu.SemaphoreType.DMA,
    ],
)


def _sc_e_body(row_h, col_h, ev_h, feat_h, outp_h,
               gb0, gb1, gb2, gb3, rb0, rb1, rb2, rb3, eb0, eb1, eb2, eb3,
               cb0, cb1, cb2, cb3, out_sh,
               gsem0, gsem1, gsem2, gsem3, ssem0, ssem1, ssem2, ssem3,
               csem0, csem1, csem2, csem3):
    cid = lax.axis_index("c")
    sid = lax.axis_index("s")
    wid = sid * NC + cid
    base = wid * EPW
    gbufs = (gb0, gb1, gb2, gb3)
    rbufs = (rb0, rb1, rb2, rb3)
    ebufs = (eb0, eb1, eb2, eb3)
    cbufs = (cb0, cb1, cb2, cb3)
    gsems = (gsem0, gsem1, gsem2, gsem3)
    ssems = (ssem0, ssem1, ssem2, ssem3)
    csems = (csem0, csem1, csem2, csem3)

    def zrow(i, _):
        for cb in range(8):
            gb0[i, pl.ds(cb * 16, 16)] = jnp.zeros((16,), jnp.float32)
        return _
    lax.fori_loop(0, 16, zrow, None)
    rbase = sid * RPT

    def zcopy(z, _):
        pltpu.sync_copy(gb0.at[pl.ds(0, 16)],
                        out_sh.at[pl.ds(rbase + z * 16, 16)])
        return _
    lax.fori_loop(0, RPT // 16, zcopy, None)

    @pl.when(sid == NS - 1)
    def _():
        pltpu.sync_copy(gb0.at[pl.ds(0, 16)],
                        out_sh.at[pl.ds(NS * RPT, N - NS * RPT)])
    plsc.subcore_barrier()

    def fetch_col(c, sl):
        pltpu.async_copy(col_h.at[pl.ds(base + c * CH, CH)], cbufs[sl],
                         csems[sl])

    def drain_col(sl):
        pltpu.make_async_copy(
            col_h.at[pl.ds(0, CH)], cbufs[sl], csems[sl]).wait()

    def issue_gather(c, sl):
        drain_col(sl)
        pltpu.async_copy(feat_h.at[cbufs[sl]], gbufs[sl], gsems[sl])
        pltpu.async_copy(row_h.at[pl.ds(base + c * CH, CH)], rbufs[sl],
                         gsems[sl])
        pltpu.async_copy(ev_h.at[pl.ds(base + c * CH, CH)], ebufs[sl],
                         gsems[sl])

    def drain_gather(sl):
        pltpu.make_async_copy(
            feat_h.at[pl.ds(0, CH)], gbufs[sl], gsems[sl]).wait()
        pltpu.make_async_copy(
            row_h.at[pl.ds(0, CH)], rbufs[sl], gsems[sl]).wait()
        pltpu.make_async_copy(
            ev_h.at[pl.ds(0, CH)], ebufs[sl], gsems[sl]).wait()

    def drain_scatter(sl):
        pltpu.make_async_copy(
            feat_h.at[pl.ds(0, CH)], gbufs[sl], ssems[sl]).wait()

    def process(c, sl):
        nsl = (sl + 3) % 4

        @pl.when(c >= 1)
        def _():
            drain_scatter(nsl)

        @pl.when(c + 3 < NCHUNK)
        def _():
            issue_gather(c + 3, nsl)

        drain_gather(sl)

        @pl.when(c + 4 < NCHUNK)
        def _():
            fetch_col(c + 4, sl)

        gb = gbufs[sl]
        rb = rbufs[sl]
        eb = ebufs[sl]

        def kstep(k, _):
            av16 = eb[pl.ds(k * 16, 16)]
            for ri in range(16):
                att = av16[ri]
                rr = k * 16 + ri
                for cb in range(8):
                    csl = pl.ds(cb * 16, 16)
                    gb[rr, csl] = gb[rr, csl] * att
            ridx = rb[pl.ds(k * 16, 16)]
            pltpu.async_copy(gb.at[pl.ds(k * 16, 16)], out_sh.at[ridx],
                             ssems[sl], add=True)
            return _
        lax.fori_loop(0, CH // 16, kstep, None)

    for c0 in range(4):
        fetch_col(c0, c0)
    issue_gather(0, 0)
    issue_gather(1, 1)
    issue_gather(2, 2)

    def outer(gg, _):
        process(gg * 4, 0)
        process(gg * 4 + 1, 1)
        process(gg * 4 + 2, 2)
        process(gg * 4 + 3, 3)
        return _
    lax.fori_loop(0, NCHUNK // 4, outer, None)
    process(NCHUNK - 1, 0)

    drain_scatter(0)
    plsc.subcore_barrier()
    pltpu.sync_copy(out_sh.at[pl.ds(rbase, RPT)],
                    outp_h.at[cid, pl.ds(rbase, RPT)])

    @pl.when(sid == NS - 1)
    def _():
        pltpu.sync_copy(out_sh.at[pl.ds(NS * RPT, N - NS * RPT)],
                        outp_h.at[cid, pl.ds(NS * RPT, N - NS * RPT)])


_sc_e = pl.kernel(
    _sc_e_body,
    out_type=jax.ShapeDtypeStruct((NC, N, D), jnp.float32),
    mesh=_mesh,
    compiler_params=pltpu.CompilerParams(needs_layout_passes=False),
    scratch_types=(
        [pltpu.VMEM((CH, D), jnp.float32)] * 4
        + [pltpu.VMEM((CH,), jnp.int32)] * 4
        + [pltpu.VMEM((CH,), jnp.float32)] * 4
        + [pltpu.VMEM((CH,), jnp.int32)] * 4
        + [pltpu.VMEM_SHARED((N, D), jnp.float32)]
        + [pltpu.SemaphoreType.DMA] * 12
    ),
)


_BN = 1000


def _proj_body(x_ref, w_ref, b_ref, aw_ref, ab_ref, feat_ref, a12_ref):
    x = x_ref[...]
    feat = lax.dot_general(x, w_ref[...], (((1,), (1,)), ((), ())),
                           preferred_element_type=jnp.float32)
    feat = feat + b_ref[...]
    feat_ref[...] = feat
    a12_ref[...] = lax.dot_general(feat, aw_ref[...], (((1,), (0,)), ((), ())),
                                   preferred_element_type=jnp.float32) + ab_ref[...]


_proj = pl.pallas_call(
    _proj_body,
    grid=(N // _BN,),
    in_specs=[
        pl.BlockSpec((_BN, D), lambda i: (i, 0)),
        pl.BlockSpec((D, D), lambda i: (0, 0)),
        pl.BlockSpec((1, D), lambda i: (0, 0)),
        pl.BlockSpec((D, 8), lambda i: (0, 0)),
        pl.BlockSpec((1, 8), lambda i: (0, 0)),
    ],
    out_specs=[
        pl.BlockSpec((_BN, D), lambda i: (i, 0)),
        pl.BlockSpec((_BN, 8), lambda i: (i, 0)),
    ],
    out_shape=[
        jax.ShapeDtypeStruct((N, D), jnp.float32),
        jax.ShapeDtypeStruct((N, 8), jnp.float32),
    ],
)


def _make_reduce(op):
    def body(p_ref, o_ref):
        o_ref[...] = jnp.broadcast_to(op(p_ref[...], axis=0, keepdims=True),
                                      (8, N))
    return pl.pallas_call(
        body,
        out_shape=jax.ShapeDtypeStruct((8, N), jnp.float32),
    )


_maxred = _make_reduce(jnp.max)
_sumred = _make_reduce(jnp.sum)


def _add_body(p_ref, sp_ref, o_ref):
    s = jnp.sum(sp_ref[...], axis=0)
    s = jnp.where(s == 0, jnp.float32(1), s)
    o_ref[...] = (p_ref[0] + p_ref[1]) / s[:, None]


_final_add = pl.pallas_call(
    _add_body,
    out_shape=jax.ShapeDtypeStruct((N, D), jnp.float32),
)


def kernel(features, adj_indices, W, b, a1_w, a1_b, a2_w, a2_b):
    row = adj_indices[0]
    col = adj_indices[1]
    aw = jnp.pad(jnp.stack([a1_w, a2_w], axis=1), ((0, 0), (0, 6)))
    ab = jnp.pad(jnp.stack([jnp.asarray(a1_b, jnp.float32),
                            jnp.asarray(a2_b, jnp.float32)])[None, :],
                 ((0, 0), (0, 6)))
    feat, a12 = _proj(features, W, b[None, :], aw, ab)
    a1 = a12[:, 0]
    a2 = a12[:, 1]
    v, mpart = _sc_a(row, col, a1, a2)
    ev, spart = _sc_c(row, v, mpart)
    outp = _sc_e(row, col, ev, feat)
    return _final_add(outp, spart.reshape(NW, N))
